# Initial kernel scaffold; baseline (speedup 1.0000x reference)
import jax
import jax.numpy as jnp
from jax.experimental import pallas as pl


def kernel(x, emb_weight):
    raise NotImplementedError("write your pallas kernel here")


# speedup vs baseline: 3.4008x; 3.4008x over previous
import functools

import jax
import jax.numpy as jnp
from jax import lax
from jax.experimental import pallas as pl
from jax.experimental.pallas import tpu as pltpu
from jax.experimental.pallas import tpu_sc as plsc

D_MODEL = 64
ROW = 128
G = 8


def _emb_sc(idx2d, table, n_rows_per_worker):
    n_rows_total = idx2d.shape[0]
    b_total = n_rows_total * ROW
    info = plsc.get_sparse_core_info()
    nc, ns = info.num_cores, info.num_subcores
    groups_per_worker = n_rows_per_worker // G

    mesh = plsc.VectorSubcoreMesh(core_axis_name="c", subcore_axis_name="s")

    @functools.partial(
        pl.kernel,
        mesh=mesh,
        compiler_params=pltpu.CompilerParams(use_tc_tiling_on_sc=False),
        out_type=jax.ShapeDtypeStruct((b_total, D_MODEL), jnp.float32),
        scratch_types=[
            pltpu.VMEM((G, ROW), jnp.int32),
            pltpu.VMEM((G * ROW, D_MODEL), jnp.float32),
            pltpu.SemaphoreType.DMA,
        ],
    )
    def k(table_hbm, idx_hbm, out_hbm, idx_v, rows_v, sem):
        wid = lax.axis_index("s") * nc + lax.axis_index("c")
        row_base = wid * n_rows_per_worker

        def body(g, _):
            r = row_base + g * G
            pltpu.sync_copy(idx_hbm.at[pl.ds(r, G)], idx_v)
            cps = [
                pltpu.async_copy(
                    table_hbm.at[idx_v.at[j]],
                    rows_v.at[pl.ds(j * ROW, ROW)],
                    sem,
                )
                for j in range(G)
            ]
            for cp in cps:
                cp.wait()
            pltpu.sync_copy(rows_v, out_hbm.at[pl.ds(r * ROW, G * ROW)])
            return ()

        lax.fori_loop(0, groups_per_worker, body, ())

    return k(table, idx2d)


def kernel(x, emb_weight):
    b, t = x.shape
    b_total = b * t
    n_rows = b_total // ROW
    info = plsc.get_sparse_core_info()
    nw = info.num_cores * info.num_subcores
    n_rows_per_worker = n_rows // nw
    assert n_rows_per_worker * nw == n_rows and n_rows_per_worker % G == 0

    idx2d = x.reshape(n_rows, ROW).astype(jnp.int32)
    out = _emb_sc(idx2d, emb_weight, n_rows_per_worker)
    return out.reshape(b, t, D_MODEL)
